# Initial kernel scaffold; baseline (speedup 1.0000x reference)
import jax
import jax.numpy as jnp
from jax.experimental import pallas as pl


def kernel(input, mask_idx, emb_table):
    raise NotImplementedError("write your pallas kernel here")


# speedup vs baseline: 4.5003x; 4.5003x over previous
import functools

import jax
import jax.numpy as jnp
from jax import lax
from jax.experimental import pallas as pl
from jax.experimental.pallas import tpu as pltpu
from jax.experimental.pallas import tpu_sc as plsc

_INFO = plsc.get_sparse_core_info()
_NC, _NS = _INFO.num_cores, _INFO.num_subcores
_NW = _NC * _NS

_N = 4096 * 50
_D = 64
_BPW = _N // _NW
_C = 640
_NCH = _BPW // _C


def _make_lookup():
  mesh = plsc.VectorSubcoreMesh(core_axis_name="c", subcore_axis_name="s")

  @functools.partial(
      pl.kernel,
      out_type=jax.ShapeDtypeStruct((_N, _D), jnp.float32),
      mesh=mesh,
      scratch_types=[
          pltpu.VMEM((_C,), jnp.int32),
          pltpu.VMEM((_C, _D), jnp.float32),
          pltpu.SemaphoreType.DMA,
      ],
      compiler_params=pltpu.CompilerParams(use_tc_tiling_on_sc=False),
  )
  def lookup(table_hbm, idx_hbm, out_hbm, idx_v, rows_v, sem):
    wid = lax.axis_index("s") * _NC + lax.axis_index("c")
    base = wid * _BPW
    for j in range(_NCH):
      pltpu.sync_copy(idx_hbm.at[wid, j], idx_v)
      pltpu.async_copy(table_hbm.at[idx_v], rows_v, sem).wait()
      pltpu.sync_copy(rows_v, out_hbm.at[pl.ds(base + j * _C, _C)])

  return lookup


_LOOKUP = _make_lookup()


@jax.jit
def kernel(input, mask_idx, emb_table):
  del input
  batch, hist = mask_idx.shape
  idx = mask_idx.astype(jnp.int32).reshape(_NW, _NCH, _C)
  out = _LOOKUP(emb_table, idx)
  return out.reshape(batch, hist, _D)


# speedup vs baseline: 4.6604x; 1.0356x over previous
import functools

import jax
import jax.numpy as jnp
from jax import lax
from jax.experimental import pallas as pl
from jax.experimental.pallas import tpu as pltpu
from jax.experimental.pallas import tpu_sc as plsc

_INFO = plsc.get_sparse_core_info()
_NC, _NS = _INFO.num_cores, _INFO.num_subcores
_NW = _NC * _NS

_N = 4096 * 50
_D = 64
_BPW = _N // _NW
_C = 640
_NCH = _BPW // _C


def _make_lookup():
  mesh = plsc.VectorSubcoreMesh(core_axis_name="c", subcore_axis_name="s")

  @functools.partial(
      pl.kernel,
      out_type=jax.ShapeDtypeStruct((_N, _D), jnp.float32),
      mesh=mesh,
      scratch_types=[
          pltpu.VMEM((_NCH, _C), jnp.int32),
          pltpu.VMEM((_C, _D), jnp.float32),
          pltpu.VMEM((_C, _D), jnp.float32),
          pltpu.SemaphoreType.DMA,
          pltpu.SemaphoreType.DMA,
          pltpu.SemaphoreType.DMA,
          pltpu.SemaphoreType.DMA,
      ],
      compiler_params=pltpu.CompilerParams(use_tc_tiling_on_sc=False),
  )
  def lookup(table_hbm, idx_hbm, out_hbm, idx_v, rows0, rows1, g0, g1, p0, p1):
    wid = lax.axis_index("s") * _NC + lax.axis_index("c")
    base = wid * _BPW
    rows = (rows0, rows1)
    gsem = (g0, g1)
    psem = (p0, p1)

    pltpu.sync_copy(idx_hbm.at[wid], idx_v)

    gets = [None, None]
    puts = [None, None]
    gets[0] = pltpu.async_copy(table_hbm.at[idx_v.at[0]], rows[0], gsem[0])
    for j in range(1, _NCH):
      b = j % 2
      if puts[b] is not None:
        puts[b].wait()
      gets[b] = pltpu.async_copy(table_hbm.at[idx_v.at[j]], rows[b], gsem[b])
      pb = (j - 1) % 2
      gets[pb].wait()
      puts[pb] = pltpu.async_copy(
          rows[pb], out_hbm.at[pl.ds(base + (j - 1) * _C, _C)], psem[pb])
    lb = (_NCH - 1) % 2
    gets[lb].wait()
    puts[lb] = pltpu.async_copy(
        rows[lb], out_hbm.at[pl.ds(base + (_NCH - 1) * _C, _C)], psem[lb])
    puts[1 - lb].wait()
    puts[lb].wait()

  return lookup


_LOOKUP = _make_lookup()


@jax.jit
def kernel(input, mask_idx, emb_table):
  del input
  batch, hist = mask_idx.shape
  idx = mask_idx.astype(jnp.int32).reshape(_NW, _NCH, _C)
  out = _LOOKUP(emb_table, idx)
  return out.reshape(batch, hist, _D)
